# baseline (speedup 1.0000x reference)
import functools

import jax
import jax.numpy as jnp
import numpy as np
from jax import lax
from jax.experimental import pallas as pl
from jax.experimental.pallas import tpu as pltpu
from jax.experimental.pallas import tpu_sc as plsc

N = 100000
K = 16
L = 16
NC = 2
NS = 16
NCT = 8
CN = 128
CG = CN // L
CE = CN * K
NSLOT = -(-N // CN)
TAIL_NB = N - CN
USC = NSLOT // NC
ROUNDS = -(-USC // NCT)
MAXA = 4

DT = np.float32(0.01)
EPS = np.float32(1e-14)
GROUND = np.float32(-2.0)
REBOUND = np.float32(0.1)
GRAV_Y = np.float32(-9.8)
LN10 = np.float32(2.302585092994046)


def _rsqrt(x):
    i = lax.bitcast_convert_type(x, jnp.int32)
    i = np.int32(0x5F3759DF) - lax.shift_right_logical(i, 1)
    y = lax.bitcast_convert_type(i, jnp.float32)
    for _ in range(3):
        y = y * (np.float32(1.5) - np.float32(0.5) * x * y * y)
    return y


_mesh = plsc.VectorSubcoreMesh(core_axis_name="c", subcore_axis_name="s")


@functools.partial(
    pl.kernel,
    out_type=jax.ShapeDtypeStruct((N * 6,), jnp.float32),
    mesh=_mesh,
    compiler_params=pltpu.CompilerParams(needs_layout_passes=False),
    scratch_types=[
        pltpu.VMEM((N,), jnp.float32),
        pltpu.VMEM((MAXA * CE,), jnp.int32),
        pltpu.VMEM((CE,), jnp.float32),
        pltpu.VMEM((CE,), jnp.float32),
        pltpu.VMEM((CE,), jnp.float32),
        pltpu.VMEM((CE,), jnp.float32),
        pltpu.VMEM((CN * 3,), jnp.float32),
        pltpu.VMEM((CN * 3,), jnp.float32),
        pltpu.VMEM((CE,), jnp.float32),
        pltpu.VMEM((CE,), jnp.float32),
        pltpu.VMEM((CN,), jnp.float32),
        pltpu.VMEM((CN * 6,), jnp.float32),
        pltpu.VMEM_SHARED((2 * NCT * 3 * CE,), jnp.float32),
        pltpu.SemaphoreType.DMA,
        pltpu.SemaphoreType.DMA,
        pltpu.SemaphoreType.DMA,
        pltpu.SemaphoreType.DMA,
        pltpu.SemaphoreType.DMA,
    ],
)
def _sc_step(xs, ys, zs, xyzf, velf, olf, gkf, gm, knnf, out,
             col_v, idx_v, gout_v, rx_v, ry_v, rz_v,
             own_v, vel_v, ol_v, gk_v, gm_v, out_v, G,
             ksem, gwsem, rbsem, lsem, osem):
    cid = lax.axis_index("c")
    sid = lax.axis_index("s")
    ii = lax.iota(jnp.int32, L)

    is_gather = sid >= NCT
    g = sid - NCT
    col = jnp.where(g < 3, 0, jnp.where(g < 6, 1, 2))
    tmem = jnp.where(g < 3, g, jnp.where(g < 6, g - 3, g - 6))
    tsz = jnp.where(g < 6, 3, 2)

    def node_base(t):
        nb = jnp.where(t == NSLOT - 1, TAIL_NB, t * CN)
        return pl.multiple_of(nb, 32)

    def region(buf, q, c):
        off = ((buf * NCT + q) * 3 + c) * CE
        return pl.multiple_of(off, CE)

    @pl.when(is_gather)
    def _():
        @pl.when(col == 0)
        def _():
            pltpu.sync_copy(xs, col_v)

        @pl.when(col == 1)
        def _():
            pltpu.sync_copy(ys, col_v)

        @pl.when(col == 2)
        def _():
            pltpu.sync_copy(zs, col_v)

    def gather_round(r):
        buf = r % 2

        def assign(a):
            q = tmem + a * tsz
            u = r * NCT + q
            return q, jnp.logical_and(q < NCT, u < USC)

        for a in range(MAXA):
            q, valid = assign(a)

            @pl.when(valid)
            def _():
                t = (r * NCT + q) * NC + cid
                nb = node_base(t)
                pltpu.make_async_copy(
                    knnf.at[pl.ds(nb * K, CE)],
                    idx_v.at[pl.ds(a * CE, CE)], ksem).start()

        for a in range(MAXA):
            q, valid = assign(a)

            @pl.when(valid)
            def _():
                pltpu.make_async_copy(
                    knnf.at[pl.ds(0, CE)],
                    idx_v.at[pl.ds(0, CE)], ksem).wait()

        for a in range(MAXA):
            q, valid = assign(a)

            @pl.when(valid)
            def _():
                def gather_body(it, u):
                    base = it * (4 * L)
                    for w in range(4):
                        pos = ii + (base + w * L)
                        kv = plsc.load_gather(idx_v, [pos + a * CE])
                        vals = plsc.load_gather(col_v, [kv])
                        plsc.store_scatter(gout_v, [pos], vals)
                    return u

                lax.fori_loop(0, CE // (4 * L), gather_body, 0)
                cp = pltpu.make_async_copy(
                    gout_v, G.at[pl.ds(region(buf, q, col), CE)], gwsem)
                cp.start()
                cp.wait()

    def compute_round(r):
        buf = r % 2
        t = (r * NCT + sid) * NC + cid
        nb = node_base(t)

        rbs = [
            pltpu.make_async_copy(
                G.at[pl.ds(region(buf, sid, 0), CE)], rx_v, rbsem),
            pltpu.make_async_copy(
                G.at[pl.ds(region(buf, sid, 1), CE)], ry_v, rbsem),
            pltpu.make_async_copy(
                G.at[pl.ds(region(buf, sid, 2), CE)], rz_v, rbsem),
        ]
        lin = [
            pltpu.make_async_copy(
                xyzf.at[pl.ds(nb * 3, CN * 3)], own_v, lsem),
            pltpu.make_async_copy(
                velf.at[pl.ds(nb * 3, CN * 3)], vel_v, lsem),
            pltpu.make_async_copy(olf.at[pl.ds(nb * K, CE)], ol_v, lsem),
            pltpu.make_async_copy(gkf.at[pl.ds(nb * K, CE)], gk_v, lsem),
            pltpu.make_async_copy(gm.at[pl.ds(nb, CN)], gm_v, lsem),
        ]
        for cp in rbs:
            cp.start()
        for cp in lin:
            cp.start()
        for cp in rbs:
            cp.wait()
        for cp in lin:
            cp.wait()

        @pl.when(r >= 1)
        def _():
            pltpu.make_async_copy(
                out_v, out.at[pl.ds(nb * 6, CN * 6)], osem).wait()

        def group_body(gr, u):
            rr = gr * L + ii
            r3 = rr * 3
            ox = plsc.load_gather(own_v, [r3])
            oy = plsc.load_gather(own_v, [r3 + 1])
            oz = plsc.load_gather(own_v, [r3 + 2])
            vx = plsc.load_gather(vel_v, [r3])
            vy = plsc.load_gather(vel_v, [r3 + 1])
            vz = plsc.load_gather(vel_v, [r3 + 2])
            mlg = plsc.load_gather(gm_v, [rr])

            ax = jnp.zeros((L,), jnp.float32)
            ay = jnp.zeros((L,), jnp.float32)
            az = jnp.zeros((L,), jnp.float32)
            rk = rr * K
            for j in range(K):
                flat = rk + j
                nx = plsc.load_gather(rx_v, [flat])
                ny = plsc.load_gather(ry_v, [flat])
                nz = plsc.load_gather(rz_v, [flat])
                olj = plsc.load_gather(ol_v, [flat])
                kj = plsc.load_gather(gk_v, [flat])
                dx = nx - ox
                dy = ny - oy
                dz = nz - oz
                d2 = dx * dx + dy * dy + dz * dz + EPS
                rinv = _rsqrt(d2)
                dist = d2 * rinv
                st = dist - olj
                kl = jnp.exp(LN10 * kj)
                aa = jnp.abs(st) + EPS
                sq = aa * _rsqrt(aa)
                fm = kl * jnp.sign(st) * sq
                coef = fm * rinv
                ax = ax + coef * dx
                ay = ay + coef * dy
                az = az + coef * dz

            invm = jnp.exp(-LN10 * mlg)
            vnx = vx + (ax * invm) * DT
            vny = vy + (ay * invm + GRAV_Y) * DT
            vnz = vz + (az * invm) * DT
            xnx = ox + vnx * DT
            xny = oy + vny * DT
            xnz = oz + vnz * DT
            below = xny < GROUND
            xny = jnp.where(below, GROUND, xny)
            vny = jnp.where(below, -vny * REBOUND, vny)

            r6 = rr * 6
            plsc.store_scatter(out_v, [r6], xnx)
            plsc.store_scatter(out_v, [r6 + 1], xny)
            plsc.store_scatter(out_v, [r6 + 2], xnz)
            plsc.store_scatter(out_v, [r6 + 3], vnx)
            plsc.store_scatter(out_v, [r6 + 4], vny)
            plsc.store_scatter(out_v, [r6 + 5], vnz)
            return u

        lax.fori_loop(0, CG, group_body, 0)
        pltpu.make_async_copy(
            out_v, out.at[pl.ds(nb * 6, CN * 6)], osem).start()

    @pl.when(is_gather)
    def _():
        gather_round(0)

    def round_body(r, carry):
        plsc.subcore_barrier()

        @pl.when(jnp.logical_and(jnp.logical_not(is_gather),
                                 r * NCT + sid < USC))
        def _():
            compute_round(r)

        @pl.when(jnp.logical_and(is_gather, r + 1 < ROUNDS))
        def _():
            gather_round(r + 1)

        return carry

    lax.fori_loop(0, ROUNDS, round_body, 0)

    @pl.when(jnp.logical_not(is_gather))
    def _():
        pltpu.make_async_copy(
            out_v, out.at[pl.ds(0, CN * 6)], osem).wait()


def kernel(xyz, velocity, origin_len, global_k, global_m, knn_index):
    xs = xyz[:, 0]
    ys = xyz[:, 1]
    zs = xyz[:, 2]
    xyzf = xyz.reshape(N * 3)
    velf = velocity.reshape(N * 3)
    olf = origin_len.reshape(N * K)
    gkf = global_k.reshape(N * K)
    knnf = knn_index.astype(jnp.int32).reshape(N * K)
    outf = _sc_step(xs, ys, zs, xyzf, velf, olf, gkf,
                    global_m.astype(jnp.float32), knnf)
    return outf.reshape(N, 6)
